# Initial kernel scaffold; baseline (speedup 1.0000x reference)
import jax
import jax.numpy as jnp
from jax.experimental import pallas as pl


def kernel(x, mask_bool, temp, w_sal, b_sal, w_gate, b_gate, scale):
    raise NotImplementedError("write your pallas kernel here")


# speedup vs baseline: 2.5414x; 2.5414x over previous
import jax
import jax.numpy as jnp
from jax.experimental import pallas as pl
from jax.experimental.pallas import tpu as pltpu

B, T, D, H = 128, 2048, 512, 8
HD = D // H
BB = 4


def _sal_kernel(x_ref, mb_ref, w_ref, beff_ref, wg_ref, bg_ref, scale_ref,
                vec_ref, uexp_ref, uh_ref):
    lane = jax.lax.broadcasted_iota(jnp.int32, (H, D), 1)
    row = jax.lax.broadcasted_iota(jnp.int32, (H, D), 0)
    bd = (lane // HD == row).astype(jnp.float32)
    w = w_ref[...]
    for b in range(BB):
        x = x_ref[b]
        s = jax.lax.dot_general(x, w, (((1,), (0,)), ((), ())),
                                preferred_element_type=jnp.float32)
        safe = s.T + beff_ref[...] + mb_ref[0, b][None, :]
        m = jnp.max(safe, axis=1, keepdims=True)
        p = jnp.exp(safe - m)
        l = jnp.sum(p, axis=1, keepdims=True)
        valid = (m > -1e8).astype(jnp.float32)
        pooled = jax.lax.dot_general(p, x, (((1,), (0,)), ((), ())),
                                     preferred_element_type=jnp.float32)
        vec_h = pooled / (l + 1e-6) * valid
        g = jnp.sum(vec_h * wg_ref[...], axis=1, keepdims=True) + bg_ref[0, 0]
        u = jax.nn.sigmoid(g) * valid
        vec = jnp.sum(vec_h * bd, axis=0, keepdims=True)
        ss = jnp.sum(vec * vec, axis=1, keepdims=True)
        inv = jax.lax.rsqrt(ss / D + 1e-6)
        vec_ref[0, b] = (vec * inv * scale_ref[...])[0]
        uexp_ref[0, b] = jnp.sum(u * bd, axis=0)
        uh_ref[0, b] = u[:, 0]


def kernel(x, mask_bool, temp, w_sal, b_sal, w_gate, b_gate, scale):
    temperature = jax.nn.softplus(temp) + 0.3
    w_eff = (w_sal / temperature[None, :]).astype(jnp.float32)
    b_eff = (b_sal / temperature).reshape(H, 1).astype(jnp.float32)
    maskbias = jnp.where(mask_bool, 0.0, -1e9).astype(jnp.float32)
    maskbias = maskbias.reshape(B // BB, BB, T)
    wg_bd = (jnp.eye(H, dtype=jnp.float32)[:, :, None]
             * w_gate[:, 0][None, None, :]).reshape(H, D)
    bg = b_gate.reshape(1, 1).astype(jnp.float32)
    scale_row = scale.reshape(1, D).astype(jnp.float32)

    grid = (B // BB,)
    vec3, uexp3, uh3 = pl.pallas_call(
        _sal_kernel,
        grid=grid,
        in_specs=[
            pl.BlockSpec((BB, T, D), lambda i: (i, 0, 0)),
            pl.BlockSpec((1, BB, T), lambda i: (i, 0, 0)),
            pl.BlockSpec((D, H), lambda i: (0, 0)),
            pl.BlockSpec((H, 1), lambda i: (0, 0)),
            pl.BlockSpec((H, D), lambda i: (0, 0)),
            pl.BlockSpec((1, 1), lambda i: (0, 0)),
            pl.BlockSpec((1, D), lambda i: (0, 0)),
        ],
        out_specs=[
            pl.BlockSpec((1, BB, D), lambda i: (i, 0, 0)),
            pl.BlockSpec((1, BB, D), lambda i: (i, 0, 0)),
            pl.BlockSpec((1, BB, H), lambda i: (i, 0, 0)),
        ],
        out_shape=[
            jax.ShapeDtypeStruct((B // BB, BB, D), jnp.float32),
            jax.ShapeDtypeStruct((B // BB, BB, D), jnp.float32),
            jax.ShapeDtypeStruct((B // BB, BB, H), jnp.float32),
        ],
        compiler_params=pltpu.CompilerParams(
            dimension_semantics=("arbitrary",),
            vmem_limit_bytes=48 * 1024 * 1024,
        ),
        name="salience_write_head",
    )(x, maskbias, w_eff, b_eff, wg_bd, bg, scale_row)

    return (vec3.reshape(B, D), uexp3.reshape(B, D), uh3.reshape(B, H))


# speedup vs baseline: 3.6899x; 1.4519x over previous
import jax
import jax.numpy as jnp
from jax.experimental import pallas as pl
from jax.experimental.pallas import tpu as pltpu

B, T, D, H = 128, 2048, 512, 8
HD = D // H
BB = 4


def _sal_kernel(x_ref, mb_ref, w_ref, beff_ref, wg_ref, bg_ref, scale_ref,
                vec_ref, uexp_ref, uh_ref):
    lane = jax.lax.broadcasted_iota(jnp.int32, (H, D), 1)
    row = jax.lax.broadcasted_iota(jnp.int32, (H, D), 0)
    bd = (lane // HD == row).astype(jnp.float32)
    w = w_ref[...]
    x_all = x_ref[...].reshape(BB * T, D)
    s = jax.lax.dot_general(x_all, w, (((1,), (0,)), ((), ())),
                            preferred_element_type=jnp.float32)
    safe_all = s.T + beff_ref[...] + mb_ref[0]
    for b in range(BB):
        x = x_ref[b]
        safe = safe_all[:, b * T:(b + 1) * T]
        m = jnp.max(safe, axis=1, keepdims=True)
        p = jnp.exp(safe - m)
        l = jnp.sum(p, axis=1, keepdims=True)
        valid = (m > -1e8).astype(jnp.float32)
        pooled = jax.lax.dot_general(p, x, (((1,), (0,)), ((), ())),
                                     preferred_element_type=jnp.float32)
        vec_h = pooled / (l + 1e-6) * valid
        g = jnp.sum(vec_h * wg_ref[...], axis=1, keepdims=True) + bg_ref[0, 0]
        u = jax.nn.sigmoid(g) * valid
        vec = jnp.sum(vec_h * bd, axis=0, keepdims=True)
        ss = jnp.sum(vec * vec, axis=1, keepdims=True)
        inv = jax.lax.rsqrt(ss / D + 1e-6)
        vec_ref[0, b] = (vec * inv * scale_ref[...])[0]
        uexp_ref[0, b] = jnp.sum(u * bd, axis=0)
        uh_ref[0, b] = u[:, 0]


def kernel(x, mask_bool, temp, w_sal, b_sal, w_gate, b_gate, scale):
    temperature = jax.nn.softplus(temp) + 0.3
    w_eff = (w_sal / temperature[None, :]).astype(jnp.float32)
    b_eff = (b_sal / temperature).reshape(H, 1).astype(jnp.float32)
    maskbias = jnp.where(mask_bool, 0.0, -1e9).astype(jnp.float32)
    maskbias = maskbias.reshape(B // BB, 1, BB * T)
    wg_bd = (jnp.eye(H, dtype=jnp.float32)[:, :, None]
             * w_gate[:, 0][None, None, :]).reshape(H, D)
    bg = b_gate.reshape(1, 1).astype(jnp.float32)
    scale_row = scale.reshape(1, D).astype(jnp.float32)

    grid = (B // BB,)
    vec3, uexp3, uh3 = pl.pallas_call(
        _sal_kernel,
        grid=grid,
        in_specs=[
            pl.BlockSpec((BB, T, D), lambda i: (i, 0, 0)),
            pl.BlockSpec((1, 1, BB * T), lambda i: (i, 0, 0)),
            pl.BlockSpec((D, H), lambda i: (0, 0)),
            pl.BlockSpec((H, 1), lambda i: (0, 0)),
            pl.BlockSpec((H, D), lambda i: (0, 0)),
            pl.BlockSpec((1, 1), lambda i: (0, 0)),
            pl.BlockSpec((1, D), lambda i: (0, 0)),
        ],
        out_specs=[
            pl.BlockSpec((1, BB, D), lambda i: (i, 0, 0)),
            pl.BlockSpec((1, BB, D), lambda i: (i, 0, 0)),
            pl.BlockSpec((1, BB, H), lambda i: (i, 0, 0)),
        ],
        out_shape=[
            jax.ShapeDtypeStruct((B // BB, BB, D), jnp.float32),
            jax.ShapeDtypeStruct((B // BB, BB, D), jnp.float32),
            jax.ShapeDtypeStruct((B // BB, BB, H), jnp.float32),
        ],
        compiler_params=pltpu.CompilerParams(
            dimension_semantics=("arbitrary",),
            vmem_limit_bytes=48 * 1024 * 1024,
        ),
        name="salience_write_head",
    )(x, maskbias, w_eff, b_eff, wg_bd, bg, scale_row)

    return (vec3.reshape(B, D), uexp3.reshape(B, D), uh3.reshape(B, H))


# speedup vs baseline: 3.6910x; 1.0003x over previous
import jax
import jax.numpy as jnp
from jax.experimental import pallas as pl
from jax.experimental.pallas import tpu as pltpu

B, T, D, H = 128, 2048, 512, 8
HD = D // H
BB = 4
NSTEPS = B // BB


def _sal_kernel(x_hbm, mb_ref, w_ref, beff_ref, wg_ref, bg_ref, scale_ref,
                vec_ref, uexp_ref, uh_ref, xbuf, sem):
    i = pl.program_id(0)
    slot = jax.lax.rem(i, 2)
    nxt = jax.lax.rem(i + 1, 2)

    @pl.when(i == 0)
    def _():
        pltpu.make_async_copy(x_hbm.at[pl.ds(i * BB, BB)],
                              xbuf.at[slot], sem.at[slot]).start()

    @pl.when(i + 1 < NSTEPS)
    def _():
        pltpu.make_async_copy(x_hbm.at[pl.ds((i + 1) * BB, BB)],
                              xbuf.at[nxt], sem.at[nxt]).start()

    pltpu.make_async_copy(x_hbm.at[pl.ds(i * BB, BB)],
                          xbuf.at[slot], sem.at[slot]).wait()

    lane = jax.lax.broadcasted_iota(jnp.int32, (H, D), 1)
    row = jax.lax.broadcasted_iota(jnp.int32, (H, D), 0)
    bd = (lane // HD == row).astype(jnp.float32)
    w = w_ref[...]
    x_blk = xbuf[slot]
    x_all = x_blk.reshape(BB * T, D)
    s = jax.lax.dot_general(x_all, w, (((1,), (0,)), ((), ())),
                            preferred_element_type=jnp.float32)
    safe_all = s.T + beff_ref[...] + mb_ref[0]
    for b in range(BB):
        x = x_blk[b]
        safe = safe_all[:, b * T:(b + 1) * T]
        m = jnp.max(safe, axis=1, keepdims=True)
        p = jnp.exp(safe - m)
        l = jnp.sum(p, axis=1, keepdims=True)
        valid = (m > -1e8).astype(jnp.float32)
        pooled = jax.lax.dot_general(p, x, (((1,), (0,)), ((), ())),
                                     preferred_element_type=jnp.float32)
        vec_h = pooled / (l + 1e-6) * valid
        g = jnp.sum(vec_h * wg_ref[...], axis=1, keepdims=True) + bg_ref[0, 0]
        u = jax.nn.sigmoid(g) * valid
        vec = jnp.sum(vec_h * bd, axis=0, keepdims=True)
        ss = jnp.sum(vec * vec, axis=1, keepdims=True)
        inv = jax.lax.rsqrt(ss / D + 1e-6)
        vec_ref[0, b] = (vec * inv * scale_ref[...])[0]
        uexp_ref[0, b] = jnp.sum(u * bd, axis=0)
        uh_ref[0, b] = u[:, 0]


def kernel(x, mask_bool, temp, w_sal, b_sal, w_gate, b_gate, scale):
    temperature = jax.nn.softplus(temp) + 0.3
    w_eff = (w_sal / temperature[None, :]).astype(jnp.float32)
    b_eff = (b_sal / temperature).reshape(H, 1).astype(jnp.float32)
    maskbias = jnp.where(mask_bool, 0.0, -1e9).astype(jnp.float32)
    maskbias = maskbias.reshape(B // BB, 1, BB * T)
    wg_bd = (jnp.eye(H, dtype=jnp.float32)[:, :, None]
             * w_gate[:, 0][None, None, :]).reshape(H, D)
    bg = b_gate.reshape(1, 1).astype(jnp.float32)
    scale_row = scale.reshape(1, D).astype(jnp.float32)

    grid = (NSTEPS,)
    vec3, uexp3, uh3 = pl.pallas_call(
        _sal_kernel,
        grid=grid,
        in_specs=[
            pl.BlockSpec(memory_space=pl.ANY),
            pl.BlockSpec((1, 1, BB * T), lambda i: (i, 0, 0)),
            pl.BlockSpec((D, H), lambda i: (0, 0)),
            pl.BlockSpec((H, 1), lambda i: (0, 0)),
            pl.BlockSpec((H, D), lambda i: (0, 0)),
            pl.BlockSpec((1, 1), lambda i: (0, 0)),
            pl.BlockSpec((1, D), lambda i: (0, 0)),
        ],
        out_specs=[
            pl.BlockSpec((1, BB, D), lambda i: (i, 0, 0)),
            pl.BlockSpec((1, BB, D), lambda i: (i, 0, 0)),
            pl.BlockSpec((1, BB, H), lambda i: (i, 0, 0)),
        ],
        out_shape=[
            jax.ShapeDtypeStruct((B // BB, BB, D), jnp.float32),
            jax.ShapeDtypeStruct((B // BB, BB, D), jnp.float32),
            jax.ShapeDtypeStruct((B // BB, BB, H), jnp.float32),
        ],
        scratch_shapes=[
            pltpu.VMEM((2, BB, T, D), jnp.float32),
            pltpu.SemaphoreType.DMA((2,)),
        ],
        compiler_params=pltpu.CompilerParams(
            dimension_semantics=("arbitrary",),
            vmem_limit_bytes=48 * 1024 * 1024,
        ),
        name="salience_write_head",
    )(x, maskbias, w_eff, b_eff, wg_bd, bg, scale_row)

    return (vec3.reshape(B, D), uexp3.reshape(B, D), uh3.reshape(B, H))
